# Initial kernel scaffold; baseline (speedup 1.0000x reference)
import jax
import jax.numpy as jnp
from jax.experimental import pallas as pl


def kernel(indices, weight):
    raise NotImplementedError("write your pallas kernel here")


# speedup vs baseline: 3.3345x; 3.3345x over previous
import functools

import jax
import jax.numpy as jnp
from jax import lax
from jax.experimental import pallas as pl
from jax.experimental.pallas import tpu as pltpu
from jax.experimental.pallas import tpu_sc as plsc

EMBEDDING_DIM = 128
NUM_CORES = 2
NUM_SUBCORES = 16
NUM_WORKERS = NUM_CORES * NUM_SUBCORES
CHUNK = 128
NBUF = 2


@functools.lru_cache(maxsize=None)
def _make_gather(n_rows: int, dim: int):
    assert n_rows % (NUM_WORKERS * CHUNK) == 0
    rows_per_w = n_rows // NUM_WORKERS
    n_chunks = rows_per_w // CHUNK
    assert n_chunks % NBUF == 0
    n_groups = n_chunks // NBUF

    mesh = plsc.VectorSubcoreMesh(
        core_axis_name="c", subcore_axis_name="s",
        num_cores=NUM_CORES, num_subcores=NUM_SUBCORES)

    @functools.partial(
        pl.kernel,
        out_type=jax.ShapeDtypeStruct((n_rows, dim), jnp.float32),
        mesh=mesh,
        scratch_types=[
            pltpu.VMEM((n_chunks, CHUNK), jnp.int32),
            pltpu.VMEM((NBUF, CHUNK, dim), jnp.float32),
            pltpu.SemaphoreType.DMA,
            pltpu.SemaphoreType.DMA,
            pltpu.SemaphoreType.DMA,
        ],
    )
    def gather_kernel(idx_hbm, table_hbm, out_hbm, idx_v, buf, isem,
                      gsem0, gsem1):
        gsem = (gsem0, gsem1)
        wid = lax.axis_index("s") * NUM_CORES + lax.axis_index("c")
        base = wid * rows_per_w

        pltpu.async_copy(idx_hbm.at[wid], idx_v, isem).wait()

        def gstart(b, c):
            pltpu.async_copy(table_hbm.at[idx_v.at[c]], buf.at[b], gsem[b])

        def gwait(b):
            pltpu.make_async_copy(
                table_hbm.at[idx_v.at[0]], buf.at[b], gsem[b]).wait()

        def swrite(b, c):
            pltpu.sync_copy(buf.at[b], out_hbm.at[pl.ds(base + c * CHUNK,
                                                        CHUNK)])

        gstart(0, 0)

        @pl.loop(0, n_groups)
        def _group(g):
            c0 = g * NBUF
            gstart(1, c0 + 1)
            gwait(0)
            swrite(0, c0)

            @pl.when(g < n_groups - 1)
            def _prefetch():
                gstart(0, c0 + 2)

            gwait(1)
            swrite(1, c0 + 1)

    return gather_kernel


def kernel(indices, weight):
    n_rows = indices.size
    dim = weight.shape[-1]
    idx_grouped = indices.reshape(NUM_WORKERS, n_rows // (NUM_WORKERS * CHUNK),
                                  CHUNK)
    out = _make_gather(n_rows, dim)(idx_grouped, weight)
    return out.reshape(indices.shape + (dim,)).astype(jnp.float32)
